# Initial kernel scaffold; baseline (speedup 1.0000x reference)
import functools

import jax
import jax.numpy as jnp
from jax import lax
from jax.experimental import pallas as pl
from jax.experimental.pallas import tpu as pltpu
from jax.experimental.pallas import tpu_sc as plsc

N = 10000
E = 320000
D_IN = 128
D_H = 64
G = 128

NPAD = 10016
NTILES = 16
NCORES = 2
NW = NCORES * NTILES
CHUNK = 128
CPW = 79
EPAD = NW * CPW * CHUNK
ROWS_PER_TILE = NPAD // NTILES


def _edge_pass(width, gather):
  mesh = plsc.VectorSubcoreMesh(core_axis_name="c", subcore_axis_name="s")

  @functools.partial(
      pl.kernel,
      mesh=mesh,
      out_type=jax.ShapeDtypeStruct((NCORES, NPAD, width), jnp.float32),
      scratch_types=[
          pltpu.VMEM((CHUNK,), jnp.int32),
          pltpu.VMEM((CHUNK,), jnp.int32),
          pltpu.VMEM((CHUNK, width), jnp.float32),
          pltpu.VMEM_SHARED((NPAD, width), jnp.float32),
          pltpu.SemaphoreType.DMA,
      ],
  )
  def k(table, src3, dst3, zeros, out, src_v, dst_v, rows_v, acc, sem):
    c = lax.axis_index("c")
    s = lax.axis_index("s")
    w = c * NTILES + s
    r0 = s * ROWS_PER_TILE
    pltpu.sync_copy(zeros.at[pl.ds(r0, ROWS_PER_TILE)],
                    acc.at[pl.ds(r0, ROWS_PER_TILE)])
    if not gather:
      pltpu.sync_copy(table, rows_v)
    plsc.subcore_barrier()

    def body(i, carry):
      pltpu.sync_copy(dst3.at[w, i], dst_v)
      if gather:
        pltpu.sync_copy(src3.at[w, i], src_v)
        pltpu.async_copy(table.at[src_v], rows_v, sem).wait()
      pltpu.sync_copy(rows_v, acc.at[dst_v], add=True)
      return carry

    lax.fori_loop(0, CPW, body, 0)
    plsc.subcore_barrier()
    pltpu.sync_copy(acc.at[pl.ds(r0, ROWS_PER_TILE)],
                    out.at[c, pl.ds(r0, ROWS_PER_TILE)])

  return k


_B = 2000


def _dinv_block(d0_ref, d1_ref):
  deg = d0_ref[0] + d1_ref[0] + 1.0
  return lax.rsqrt(deg[:, 0:1])


def _tc1(x, W1, degp):
  def body(x_ref, w_ref, d0_ref, d1_ref, g_ref):
    dinv = _dinv_block(d0_ref, d1_ref)
    h = jnp.dot(x_ref[...], w_ref[...], preferred_element_type=jnp.float32)
    g_ref[...] = h * dinv

  return pl.pallas_call(
      body,
      grid=(N // _B,),
      in_specs=[
          pl.BlockSpec((_B, D_IN), lambda i: (i, 0)),
          pl.BlockSpec((D_IN, D_H), lambda i: (0, 0)),
          pl.BlockSpec((1, _B, 16), lambda i: (0, i, 0)),
          pl.BlockSpec((1, _B, 16), lambda i: (1, i, 0)),
      ],
      out_specs=pl.BlockSpec((_B, D_H), lambda i: (i, 0)),
      out_shape=jax.ShapeDtypeStruct((N, D_H), jnp.float32),
  )(x, W1, degp, degp)


def _tc2(p, g1, degp, b1, W2):
  def body(p0_ref, p1_ref, g1_ref, d0_ref, d1_ref, b_ref, w_ref, o_ref):
    dinv = _dinv_block(d0_ref, d1_ref)
    acc = p0_ref[0] + p1_ref[0] + g1_ref[...]
    h1 = jnp.maximum(dinv * acc + b_ref[...], 0.0)
    h2 = jnp.dot(h1, w_ref[...], preferred_element_type=jnp.float32)
    o_ref[...] = h2 * dinv

  return pl.pallas_call(
      body,
      grid=(N // _B,),
      in_specs=[
          pl.BlockSpec((1, _B, D_H), lambda i: (0, i, 0)),
          pl.BlockSpec((1, _B, D_H), lambda i: (1, i, 0)),
          pl.BlockSpec((_B, D_H), lambda i: (i, 0)),
          pl.BlockSpec((1, _B, 16), lambda i: (0, i, 0)),
          pl.BlockSpec((1, _B, 16), lambda i: (1, i, 0)),
          pl.BlockSpec((1, D_H), lambda i: (0, 0)),
          pl.BlockSpec((D_H, D_H), lambda i: (0, 0)),
      ],
      out_specs=pl.BlockSpec((_B, D_H), lambda i: (i, 0)),
      out_shape=jax.ShapeDtypeStruct((N, D_H), jnp.float32),
  )(p, p, g1, degp, degp, b1, W2)


def _tc3(q, g2, degp, b2, batch2):
  nsteps = N // _B

  def body(q0_ref, q1_ref, g2_ref, d0_ref, d1_ref, b_ref, bat_ref, o_ref, cnt):
    i = pl.program_id(0)
    dinv = _dinv_block(d0_ref, d1_ref)
    h = jnp.maximum(
        dinv * (q0_ref[0] + q1_ref[0] + g2_ref[...]) + b_ref[...], 0.0)
    onehot = (bat_ref[...] ==
              lax.broadcasted_iota(jnp.int32, (_B, G), 1)).astype(jnp.float32)
    dims = (((0,), (0,)), ((), ()))
    psum = lax.dot_general(onehot, h, dims,
                           preferred_element_type=jnp.float32)
    pcnt = lax.dot_general(onehot, jnp.ones((_B, 1), jnp.float32), dims,
                           preferred_element_type=jnp.float32)

    @pl.when(i == 0)
    def _():
      o_ref[...] = psum
      cnt[...] = pcnt

    @pl.when(i > 0)
    def _():
      o_ref[...] = o_ref[...] + psum
      cnt[...] = cnt[...] + pcnt

    @pl.when(i == nsteps - 1)
    def _():
      o_ref[...] = o_ref[...] / jnp.maximum(cnt[...], 1.0)

  return pl.pallas_call(
      body,
      grid=(nsteps,),
      in_specs=[
          pl.BlockSpec((1, _B, D_H), lambda i: (0, i, 0)),
          pl.BlockSpec((1, _B, D_H), lambda i: (1, i, 0)),
          pl.BlockSpec((_B, D_H), lambda i: (i, 0)),
          pl.BlockSpec((1, _B, 16), lambda i: (0, i, 0)),
          pl.BlockSpec((1, _B, 16), lambda i: (1, i, 0)),
          pl.BlockSpec((1, D_H), lambda i: (0, 0)),
          pl.BlockSpec((_B, 1), lambda i: (i, 0)),
      ],
      out_specs=pl.BlockSpec((G, D_H), lambda i: (0, 0)),
      out_shape=jax.ShapeDtypeStruct((G, D_H), jnp.float32),
      scratch_shapes=[pltpu.VMEM((G, 1), jnp.float32)],
  )(q, q, g2, degp, degp, b2, batch2)


def kernel(x, edge_index, batch, W1, b1, W2, b2):
  src = edge_index[0].astype(jnp.int32)
  dst = edge_index[1].astype(jnp.int32)
  padlen = EPAD - E
  src3 = jnp.concatenate(
      [src, jnp.zeros((padlen,), jnp.int32)]).reshape(NW, CPW, CHUNK)
  dst3 = jnp.concatenate(
      [dst, jnp.full((padlen,), N, jnp.int32)]).reshape(NW, CPW, CHUNK)
  zeros16 = jnp.zeros((NPAD, 16), jnp.float32)
  zeros64 = jnp.zeros((NPAD, D_H), jnp.float32)
  ones16 = jnp.ones((CHUNK, 16), jnp.float32)

  deg_pass = _edge_pass(16, gather=False)
  msg_pass = _edge_pass(D_H, gather=True)

  degp = deg_pass(ones16, src3, dst3, zeros16)
  g1 = _tc1(x, W1, degp)
  p = msg_pass(g1, src3, dst3, zeros64)
  g2 = _tc2(p, g1, degp, b1.reshape(1, D_H), W2)
  q = msg_pass(g2, src3, dst3, zeros64)
  return _tc3(q, g2, degp, b2.reshape(1, D_H),
              batch.astype(jnp.int32).reshape(N, 1))


# speedup vs baseline: 9.7997x; 9.7997x over previous
import functools

import jax
import jax.numpy as jnp
from jax import lax
from jax.experimental import pallas as pl
from jax.experimental.pallas import tpu as pltpu
from jax.experimental.pallas import tpu_sc as plsc

N = 10000
E = 320000
D_IN = 128
D_H = 64
G = 128
W_PAD = 128

NPAD = 10112
NTILES = 16
NCORES = 2
NW = NCORES * NTILES
CHUNK = 128
CPW = 79
EPAD = NW * CPW * CHUNK
ROWS_PER_TILE = NPAD // NTILES


def _edge_pass(gather):
  mesh = plsc.VectorSubcoreMesh(core_axis_name="c", subcore_axis_name="s")

  @functools.partial(
      pl.kernel,
      mesh=mesh,
      out_type=jax.ShapeDtypeStruct((NCORES, NPAD, W_PAD), jnp.float32),
      scratch_types=[
          pltpu.VMEM((CHUNK,), jnp.int32),
          pltpu.VMEM((CHUNK,), jnp.int32),
          pltpu.VMEM((CHUNK, W_PAD), jnp.float32),
          pltpu.VMEM_SHARED((NPAD, W_PAD), jnp.float32),
          pltpu.SemaphoreType.DMA,
      ],
  )
  def k(table, src3, dst3, zeros, out, src_v, dst_v, rows_v, acc, sem):
    c = lax.axis_index("c")
    s = lax.axis_index("s")
    w = c * NTILES + s
    r0 = s * ROWS_PER_TILE
    pltpu.sync_copy(zeros.at[pl.ds(r0, ROWS_PER_TILE)],
                    acc.at[pl.ds(r0, ROWS_PER_TILE)])
    if not gather:
      pltpu.sync_copy(table, rows_v)
    plsc.subcore_barrier()

    def body(i, carry):
      pltpu.sync_copy(dst3.at[w, i], dst_v)
      if gather:
        pltpu.sync_copy(src3.at[w, i], src_v)
        pltpu.async_copy(table.at[src_v], rows_v, sem).wait()
      pltpu.sync_copy(rows_v, acc.at[dst_v], add=True)
      return carry

    lax.fori_loop(0, CPW, body, 0)
    plsc.subcore_barrier()
    pltpu.sync_copy(acc.at[pl.ds(r0, ROWS_PER_TILE)],
                    out.at[c, pl.ds(r0, ROWS_PER_TILE)])

  return k


_B = 2000


def _dinv_block(d0_ref, d1_ref):
  return lax.rsqrt(d0_ref[0][:, 0:1] + d1_ref[0][:, 0:1] + 1.0)


def _tc1(x, W1p, degp):
  def body(x_ref, w_ref, d0_ref, d1_ref, g_ref):
    dinv = _dinv_block(d0_ref, d1_ref)
    h = jnp.dot(x_ref[...], w_ref[...], preferred_element_type=jnp.float32)
    g_ref[...] = h * dinv

  return pl.pallas_call(
      body,
      grid=(N // _B,),
      in_specs=[
          pl.BlockSpec((_B, D_IN), lambda i: (i, 0)),
          pl.BlockSpec((D_IN, W_PAD), lambda i: (0, 0)),
          pl.BlockSpec((1, _B, W_PAD), lambda i: (0, i, 0)),
          pl.BlockSpec((1, _B, W_PAD), lambda i: (1, i, 0)),
      ],
      out_specs=pl.BlockSpec((_B, W_PAD), lambda i: (i, 0)),
      out_shape=jax.ShapeDtypeStruct((N, W_PAD), jnp.float32),
  )(x, W1p, degp, degp)


def _tc2(p, g1, degp, b1p, W2p):
  def body(p0_ref, p1_ref, g1_ref, d0_ref, d1_ref, b_ref, w_ref, o_ref):
    dinv = _dinv_block(d0_ref, d1_ref)
    acc = p0_ref[0] + p1_ref[0] + g1_ref[...]
    h1 = jnp.maximum(dinv * acc + b_ref[...], 0.0)
    h2 = jnp.dot(h1, w_ref[...], preferred_element_type=jnp.float32)
    o_ref[...] = h2 * dinv

  return pl.pallas_call(
      body,
      grid=(N // _B,),
      in_specs=[
          pl.BlockSpec((1, _B, W_PAD), lambda i: (0, i, 0)),
          pl.BlockSpec((1, _B, W_PAD), lambda i: (1, i, 0)),
          pl.BlockSpec((_B, W_PAD), lambda i: (i, 0)),
          pl.BlockSpec((1, _B, W_PAD), lambda i: (0, i, 0)),
          pl.BlockSpec((1, _B, W_PAD), lambda i: (1, i, 0)),
          pl.BlockSpec((1, W_PAD), lambda i: (0, 0)),
          pl.BlockSpec((W_PAD, W_PAD), lambda i: (0, 0)),
      ],
      out_specs=pl.BlockSpec((_B, W_PAD), lambda i: (i, 0)),
      out_shape=jax.ShapeDtypeStruct((N, W_PAD), jnp.float32),
  )(p, p, g1, degp, degp, b1p, W2p)


def _tc3(q, g2, degp, b2p, batch2):
  nsteps = N // _B

  def body(q0_ref, q1_ref, g2_ref, d0_ref, d1_ref, b_ref, bat_ref, o_ref, cnt):
    i = pl.program_id(0)
    dinv = _dinv_block(d0_ref, d1_ref)
    h = jnp.maximum(
        dinv * (q0_ref[0] + q1_ref[0] + g2_ref[...]) + b_ref[...], 0.0)
    onehot = (bat_ref[...] ==
              lax.broadcasted_iota(jnp.int32, (_B, G), 1)).astype(jnp.float32)
    dims = (((0,), (0,)), ((), ()))
    psum = lax.dot_general(onehot, h, dims,
                           preferred_element_type=jnp.float32)
    pcnt = lax.dot_general(onehot, jnp.ones((_B, 1), jnp.float32), dims,
                           preferred_element_type=jnp.float32)

    @pl.when(i == 0)
    def _():
      o_ref[...] = psum
      cnt[...] = pcnt

    @pl.when(i > 0)
    def _():
      o_ref[...] = o_ref[...] + psum
      cnt[...] = cnt[...] + pcnt

    @pl.when(i == nsteps - 1)
    def _():
      o_ref[...] = o_ref[...] / jnp.maximum(cnt[...], 1.0)

  return pl.pallas_call(
      body,
      grid=(nsteps,),
      in_specs=[
          pl.BlockSpec((1, _B, W_PAD), lambda i: (0, i, 0)),
          pl.BlockSpec((1, _B, W_PAD), lambda i: (1, i, 0)),
          pl.BlockSpec((_B, W_PAD), lambda i: (i, 0)),
          pl.BlockSpec((1, _B, W_PAD), lambda i: (0, i, 0)),
          pl.BlockSpec((1, _B, W_PAD), lambda i: (1, i, 0)),
          pl.BlockSpec((1, W_PAD), lambda i: (0, 0)),
          pl.BlockSpec((_B, 1), lambda i: (i, 0)),
      ],
      out_specs=pl.BlockSpec((G, W_PAD), lambda i: (0, 0)),
      out_shape=jax.ShapeDtypeStruct((G, W_PAD), jnp.float32),
      scratch_shapes=[pltpu.VMEM((G, 1), jnp.float32)],
  )(q, q, g2, degp, degp, b2p, batch2)


def kernel(x, edge_index, batch, W1, b1, W2, b2):
  src = edge_index[0].astype(jnp.int32)
  dst = edge_index[1].astype(jnp.int32)
  padlen = EPAD - E
  src3 = jnp.concatenate(
      [src, jnp.zeros((padlen,), jnp.int32)]).reshape(NW, CPW, CHUNK)
  dst3 = jnp.concatenate(
      [dst, jnp.full((padlen,), N, jnp.int32)]).reshape(NW, CPW, CHUNK)
  zeros = jnp.zeros((NPAD, W_PAD), jnp.float32)
  ones = jnp.ones((CHUNK, W_PAD), jnp.float32)
  W1p = jnp.pad(W1, ((0, 0), (0, W_PAD - D_H)))
  W2p = jnp.pad(W2, ((0, W_PAD - D_H), (0, W_PAD - D_H)))
  b1p = jnp.pad(b1, (0, W_PAD - D_H)).reshape(1, W_PAD)
  b2p = jnp.pad(b2, (0, W_PAD - D_H)).reshape(1, W_PAD)

  deg_pass = _edge_pass(gather=False)
  msg_pass = _edge_pass(gather=True)

  degp = deg_pass(ones, src3, dst3, zeros)
  g1 = _tc1(x, W1p, degp)
  p = msg_pass(g1, src3, dst3, zeros)
  g2 = _tc2(p, g1, degp, b1p, W2p)
  q = msg_pass(g2, src3, dst3, zeros)
  pooled = _tc3(q, g2, degp, b2p,
                batch.astype(jnp.int32).reshape(N, 1))
  return pooled[:, :D_H]
